# Initial kernel scaffold; baseline (speedup 1.0000x reference)
import jax
import jax.numpy as jnp
from jax.experimental import pallas as pl


def kernel(x, beat_info, edge_index, pitch_table, beat_table, dur_table, W_l, W_r, b):
    raise NotImplementedError("write your pallas kernel here")


# speedup vs baseline: 8.3061x; 8.3061x over previous
import functools

import jax
import jax.numpy as jnp
from jax import lax
from jax.experimental import pallas as pl
from jax.experimental.pallas import tpu as pltpu
from jax.experimental.pallas import tpu_sc as plsc

N = 50000
E = 800000
RW = 72
HALF = 25000
ROWS = 25088
NP = 2 * ROWS
EP = 819200
PT = EP // 16
CH = 1024
NCH = PT // CH
PER_SC = ROWS * RW
PER_TILE = PER_SC // 16
ZB = 2304
CR = CH // 128
BN = 512
GRID = NP // BN


def _sc_hist(src_hbm, dst_hbm, code_hbm, out_hbm,
             srcv, dstv, codes_v, addr_v, ones_v, hi_v, zbuf,
             code_sh, counts_sh):
    c = lax.axis_index("c")
    s = lax.axis_index("s")
    lo = c * HALF

    for m in range(128 // 16):
        ones_v[pl.ds(m * 16, 16)] = jnp.ones((16,), jnp.int32)
        hi_v[pl.ds(m * 16, 16)] = jnp.full((16,), 65536, jnp.int32)

    def zfill(i, _):
        zbuf[pl.ds(i * 16, 16)] = jnp.zeros((16,), jnp.int32)
        return 0
    lax.fori_loop(0, ZB // 16, zfill, 0)

    zbase = s * PER_TILE
    def zcp(i, _):
        pltpu.sync_copy(zbuf, counts_sh.at[pl.ds(zbase + i * ZB, ZB)])
        return 0
    lax.fori_loop(0, PER_TILE // ZB, zcp, 0)

    @pl.when(s == 0)
    def _():
        pltpu.sync_copy(code_hbm, code_sh)
    plsc.subcore_barrier()

    trow = s * (PT // 128)
    def chunk(g, _):
        rb = trow + g * CR
        pltpu.sync_copy(src_hbm.at[pl.ds(rb, CR)], srcv)
        pltpu.sync_copy(dst_hbm.at[pl.ds(rb, CR)], dstv)
        for j in range(CR):
            pltpu.sync_copy(code_sh.at[srcv.at[j]], codes_v.at[j])
        for i in range(CH // 16):
            j, k = i // 8, (i % 8) * 16
            cv = codes_v[j, pl.ds(k, 16)]
            dv = dstv[j, pl.ds(k, 16)]
            p = lax.shift_right_logical(cv, 7)
            dr = lax.bitwise_and(cv, 127)
            row = dv - lo
            ok = (row >= 0) & (row < HALF)
            rowt = jnp.where(ok, row, HALF)
            base = rowt * RW
            addr_v[j, pl.ds(k, 16)] = base + p
            addr_v[j + 8, pl.ds(k, 16)] = base + dr
        for j in range(8):
            pltpu.sync_copy(ones_v, counts_sh.at[addr_v.at[j]], add=True)
        for j in range(8, 16):
            pltpu.sync_copy(hi_v, counts_sh.at[addr_v.at[j]], add=True)
        return 0
    lax.fori_loop(0, NCH, chunk, 0)
    plsc.subcore_barrier()

    gout = c * PER_SC + s * PER_TILE
    pltpu.sync_copy(counts_sh.at[pl.ds(zbase, PER_TILE)],
                    out_hbm.at[pl.ds(gout, PER_TILE)])


_sc_hist_call = functools.partial(
    pl.kernel,
    mesh=plsc.VectorSubcoreMesh(core_axis_name="c", subcore_axis_name="s"),
    compiler_params=pltpu.CompilerParams(needs_layout_passes=False),
    out_type=jax.ShapeDtypeStruct((NP * RW,), jnp.int32),
    scratch_types=[
        pltpu.VMEM((CR, 128), jnp.int32),
        pltpu.VMEM((CR, 128), jnp.int32),
        pltpu.VMEM((CR, 128), jnp.int32),
        pltpu.VMEM((16, 128), jnp.int32),
        pltpu.VMEM((128,), jnp.int32),
        pltpu.VMEM((128,), jnp.int32),
        pltpu.VMEM((ZB,), jnp.int32),
        pltpu.VMEM_SHARED((N,), jnp.int32),
        pltpu.VMEM_SHARED((PER_SC,), jnp.int32),
    ],
)(_sc_hist)


def _fold(pt_ref, dt_ref, bt_ref, wl_ref, wr_ref,
          wcp_ref, wcd_ref, wp_ref, wd_ref, wb_ref):
    pt = pt_ref[:]
    dt = dt_ref[:]
    bt = bt_ref[:]
    wl = wl_ref[:]
    wr = wr_ref[:]
    z32 = jnp.zeros((72, 32), jnp.float32)
    z128 = jnp.zeros((72, 128), jnp.float32)
    wcp_ref[:] = jnp.concatenate(
        [jnp.dot(pt, wl[0:32, :], preferred_element_type=jnp.float32), z32], 1)
    wcd_ref[:] = jnp.concatenate(
        [jnp.dot(dt, wl[32:64, :], preferred_element_type=jnp.float32), z32], 1)
    wp_ref[:] = jnp.concatenate(
        [jnp.dot(pt, wr[0:32, :], preferred_element_type=jnp.float32), z32], 1)
    wd_ref[:] = jnp.concatenate(
        [jnp.dot(dt, wr[32:64, :], preferred_element_type=jnp.float32), z32], 1)
    wb_ref[:] = jnp.concatenate([z128, bt], 1)


def _fold_call(pt72, dt72, bt72, W_l, W_r):
    w = jax.ShapeDtypeStruct((72, 160), jnp.float32)
    return pl.pallas_call(
        _fold, out_shape=[w, w, w, w, w],
    )(pt72, dt72, bt72, W_l, W_r)


def _main(cnt_ref, p_ref, d_ref, bi_ref, wcp_ref, wcd_ref, wp_ref, wd_ref,
          wb_ref, b_ref, o_ref):
    ci = cnt_ref[:]
    pc = lax.bitwise_and(ci, 65535).astype(jnp.float32)
    dc = lax.shift_right_logical(ci, 16).astype(jnp.float32)
    deg = jnp.sum(pc, axis=1, keepdims=True)
    inv = 1.0 / jnp.maximum(deg, 1.0)
    iot = lax.broadcasted_iota(jnp.int32, (BN, 72), 1)
    ohp = (p_ref[:] == iot).astype(jnp.float32)
    ohd = (d_ref[:] == iot).astype(jnp.float32)
    ohb = (bi_ref[:] == iot).astype(jnp.float32)
    big = (jnp.dot(pc * inv, wcp_ref[:], preferred_element_type=jnp.float32)
           + jnp.dot(dc * inv, wcd_ref[:], preferred_element_type=jnp.float32)
           + jnp.dot(ohp, wp_ref[:], preferred_element_type=jnp.float32)
           + jnp.dot(ohd, wd_ref[:], preferred_element_type=jnp.float32)
           + jnp.dot(ohb, wb_ref[:], preferred_element_type=jnp.float32))
    left = big[:, 0:128] + b_ref[:]
    ss = jnp.sum(left * left, axis=1, keepdims=True)
    denom = jnp.maximum(jnp.sqrt(ss), 1e-12)
    left = left / denom
    left = jnp.where(left > 0, left, 0.2 * left)
    o_ref[:, 0:128] = left
    o_ref[:, 128:160] = big[:, 128:160]


def _main_call(counts, p2, d2, b2, wcp, wcd, wp, wd, wb, bias):
    full = lambda shp: pl.BlockSpec(shp, lambda i: (0, 0))
    return pl.pallas_call(
        _main,
        grid=(GRID,),
        in_specs=[
            pl.BlockSpec((BN, RW), lambda i: (i, 0)),
            pl.BlockSpec((BN, 1), lambda i: (i, 0)),
            pl.BlockSpec((BN, 1), lambda i: (i, 0)),
            pl.BlockSpec((BN, 1), lambda i: (i, 0)),
            full((72, 160)),
            full((72, 160)),
            full((72, 160)),
            full((72, 160)),
            full((72, 160)),
            full((1, 128)),
        ],
        out_specs=pl.BlockSpec((BN, 160), lambda i: (i, 0)),
        out_shape=jax.ShapeDtypeStruct((NP, 160), jnp.float32),
    )(counts, p2, d2, b2, wcp, wcd, wp, wd, wb, bias)


def _pad_ids(a):
    z = jnp.zeros((ROWS - HALF,), a.dtype)
    return jnp.concatenate([a[:HALF], z, a[HALF:], z]).reshape(NP, 1)


def kernel(x, beat_info, edge_index, pitch_table, beat_table, dur_table,
           W_l, W_r, b):
    pitch = x[:, 2]
    dur = x[:, 3]
    code = pitch * 128 + dur
    src = edge_index[0]
    dst = edge_index[1]
    srcp = jnp.concatenate([src, jnp.zeros((EP - E,), jnp.int32)]).reshape(
        EP // 128, 128)
    dstp = jnp.concatenate([dst, jnp.full((EP - E,), N, jnp.int32)]).reshape(
        EP // 128, 128)

    counts = _sc_hist_call(srcp, dstp, code).reshape(NP, RW)

    p2 = _pad_ids(pitch)
    d2 = _pad_ids(dur)
    bi2 = _pad_ids(beat_info)
    pt72 = jnp.pad(pitch_table[:66], ((0, 6), (0, 0)))
    dt72 = jnp.pad(dur_table, ((0, 6), (0, 0)))
    bt72 = jnp.pad(beat_table, ((0, 6), (0, 0)))
    wcp, wcd, wp, wd, wb = _fold_call(pt72, dt72, bt72, W_l, W_r)

    out = _main_call(counts, p2, d2, bi2, wcp, wcd, wp, wd, wb,
                     b.reshape(1, 128))
    return jnp.concatenate([out[0:HALF], out[ROWS:ROWS + HALF]], axis=0)


# speedup vs baseline: 15.8356x; 1.9065x over previous
import functools

import jax
import jax.numpy as jnp
from jax import lax
from jax.experimental import pallas as pl
from jax.experimental.pallas import tpu as pltpu
from jax.experimental.pallas import tpu_sc as plsc

N = 50000
E = 800000
RW = 72
ROWS = 25088
ALLOC = 25216
NP = 2 * ROWS
EP = 819200
PT = EP // 16
CH = 1024
NCH = PT // CH
PER_SC = ROWS * RW
PER_TILE = PER_SC // 16
ZB = 2304
CR = CH // 128
BN = 512
GRID = NP // BN


def _sc_hist(src_hbm, dst_hbm, code_hbm, out_hbm,
             srcv, dstv, codes_v, addr_v, ones_v, hi_v, zbuf,
             code_sh, counts_sh, sem_g, sem_a):
    c = lax.axis_index("c")
    s = lax.axis_index("s")
    lo = c * ROWS

    for m in range(128 // 16):
        ones_v[pl.ds(m * 16, 16)] = jnp.ones((16,), jnp.int32)
        hi_v[pl.ds(m * 16, 16)] = jnp.full((16,), 65536, jnp.int32)

    def zfill(i, _):
        zbuf[pl.ds(i * 16, 16)] = jnp.zeros((16,), jnp.int32)
        return 0
    lax.fori_loop(0, ZB // 16, zfill, 0)

    zbase = s * PER_TILE
    def zcp(i, _):
        pltpu.sync_copy(zbuf, counts_sh.at[pl.ds(zbase + i * ZB, ZB)])
        return 0
    lax.fori_loop(0, PER_TILE // ZB, zcp, 0)

    @pl.when(s == 0)
    def _():
        pltpu.sync_copy(code_hbm, code_sh)
    plsc.subcore_barrier()

    def _drain_adds():
        for j in range(8):
            pltpu.make_async_copy(ones_v, counts_sh.at[addr_v.at[j]],
                                  sem_a).wait()
        for j in range(8, 16):
            pltpu.make_async_copy(hi_v, counts_sh.at[addr_v.at[j]],
                                  sem_a).wait()

    trow = s * (PT // 128)
    def chunk(g, _):
        rb = trow + g * CR
        pltpu.sync_copy(src_hbm.at[pl.ds(rb, CR)], srcv)
        pltpu.sync_copy(dst_hbm.at[pl.ds(rb, CR)], dstv)
        gcps = [pltpu.async_copy(code_sh.at[srcv.at[j]], codes_v.at[j], sem_g)
                for j in range(CR)]
        for cp in gcps:
            cp.wait()
        @pl.when(g > 0)
        def _():
            _drain_adds()
        for i in range(CH // 16):
            j, k = i // 8, (i % 8) * 16
            cv = codes_v[j, pl.ds(k, 16)]
            dv = dstv[j, pl.ds(k, 16)]
            p = lax.shift_right_logical(cv, 7)
            dr = lax.bitwise_and(cv, 127)
            row = dv - lo
            ok = (row >= 0) & (row < ROWS)
            rowt = jnp.where(ok, row, ROWS)
            base = rowt * RW
            addr_v[j, pl.ds(k, 16)] = base + p
            addr_v[j + 8, pl.ds(k, 16)] = base + dr
        for j in range(8):
            pltpu.async_copy(ones_v, counts_sh.at[addr_v.at[j]], sem_a,
                             add=True)
        for j in range(8, 16):
            pltpu.async_copy(hi_v, counts_sh.at[addr_v.at[j]], sem_a,
                             add=True)
        return 0
    lax.fori_loop(0, NCH, chunk, 0)
    _drain_adds()
    plsc.subcore_barrier()

    gout = c * PER_SC + s * PER_TILE
    pltpu.sync_copy(counts_sh.at[pl.ds(zbase, PER_TILE)],
                    out_hbm.at[pl.ds(gout, PER_TILE)])


_sc_hist_call = functools.partial(
    pl.kernel,
    mesh=plsc.VectorSubcoreMesh(core_axis_name="c", subcore_axis_name="s"),
    compiler_params=pltpu.CompilerParams(needs_layout_passes=False),
    out_type=jax.ShapeDtypeStruct((NP * RW,), jnp.int32),
    scratch_types=[
        pltpu.VMEM((CR, 128), jnp.int32),
        pltpu.VMEM((CR, 128), jnp.int32),
        pltpu.VMEM((CR, 128), jnp.int32),
        pltpu.VMEM((16, 128), jnp.int32),
        pltpu.VMEM((128,), jnp.int32),
        pltpu.VMEM((128,), jnp.int32),
        pltpu.VMEM((ZB,), jnp.int32),
        pltpu.VMEM_SHARED((N,), jnp.int32),
        pltpu.VMEM_SHARED((ALLOC * RW,), jnp.int32),
        pltpu.SemaphoreType.DMA,
        pltpu.SemaphoreType.DMA,
    ],
)(_sc_hist)


def _fold(pt_ref, dt_ref, bt_ref, wl_ref, wr_ref,
          wcp_ref, wcd_ref, wp_ref, wd_ref, wb_ref):
    pt = pt_ref[:]
    dt = dt_ref[:]
    bt = bt_ref[:]
    wl = wl_ref[:]
    wr = wr_ref[:]
    z32 = jnp.zeros((72, 32), jnp.float32)
    z128 = jnp.zeros((72, 128), jnp.float32)
    wcp_ref[:] = jnp.concatenate(
        [jnp.dot(pt, wl[0:32, :], preferred_element_type=jnp.float32), z32], 1)
    wcd_ref[:] = jnp.concatenate(
        [jnp.dot(dt, wl[32:64, :], preferred_element_type=jnp.float32), z32], 1)
    wp_ref[:] = jnp.concatenate(
        [jnp.dot(pt, wr[0:32, :], preferred_element_type=jnp.float32), z32], 1)
    wd_ref[:] = jnp.concatenate(
        [jnp.dot(dt, wr[32:64, :], preferred_element_type=jnp.float32), z32], 1)
    wb_ref[:] = jnp.concatenate([z128, bt], 1)


def _fold_call(pt72, dt72, bt72, W_l, W_r):
    w = jax.ShapeDtypeStruct((72, 160), jnp.float32)
    return pl.pallas_call(
        _fold, out_shape=[w, w, w, w, w],
    )(pt72, dt72, bt72, W_l, W_r)


def _main(cnt_ref, p_ref, d_ref, bi_ref, wcp_ref, wcd_ref, wp_ref, wd_ref,
          wb_ref, b_ref, o_ref):
    ci = cnt_ref[:]
    pc = lax.bitwise_and(ci, 65535).astype(jnp.float32)
    dc = lax.shift_right_logical(ci, 16).astype(jnp.float32)
    deg = jnp.sum(pc, axis=1, keepdims=True)
    inv = 1.0 / jnp.maximum(deg, 1.0)
    iot = lax.broadcasted_iota(jnp.int32, (BN, 72), 1)
    ohp = (p_ref[:] == iot).astype(jnp.float32)
    ohd = (d_ref[:] == iot).astype(jnp.float32)
    ohb = (bi_ref[:] == iot).astype(jnp.float32)
    big = (jnp.dot(pc * inv, wcp_ref[:], preferred_element_type=jnp.float32)
           + jnp.dot(dc * inv, wcd_ref[:], preferred_element_type=jnp.float32)
           + jnp.dot(ohp, wp_ref[:], preferred_element_type=jnp.float32)
           + jnp.dot(ohd, wd_ref[:], preferred_element_type=jnp.float32)
           + jnp.dot(ohb, wb_ref[:], preferred_element_type=jnp.float32))
    left = big[:, 0:128] + b_ref[:]
    ss = jnp.sum(left * left, axis=1, keepdims=True)
    denom = jnp.maximum(jnp.sqrt(ss), 1e-12)
    left = left / denom
    left = jnp.where(left > 0, left, 0.2 * left)
    o_ref[:, 0:128] = left
    o_ref[:, 128:160] = big[:, 128:160]


def _main_call(counts, p2, d2, b2, wcp, wcd, wp, wd, wb, bias):
    full = lambda shp: pl.BlockSpec(shp, lambda i: (0, 0))
    return pl.pallas_call(
        _main,
        grid=(GRID,),
        in_specs=[
            pl.BlockSpec((BN, RW), lambda i: (i, 0)),
            pl.BlockSpec((BN, 1), lambda i: (i, 0)),
            pl.BlockSpec((BN, 1), lambda i: (i, 0)),
            pl.BlockSpec((BN, 1), lambda i: (i, 0)),
            full((72, 160)),
            full((72, 160)),
            full((72, 160)),
            full((72, 160)),
            full((72, 160)),
            full((1, 128)),
        ],
        out_specs=pl.BlockSpec((BN, 160), lambda i: (i, 0)),
        out_shape=jax.ShapeDtypeStruct((N, 160), jnp.float32),
    )(counts, p2, d2, b2, wcp, wcd, wp, wd, wb, bias)


def _pad_ids(a):
    return jnp.pad(a, (0, NP - N)).reshape(NP, 1)


def kernel(x, beat_info, edge_index, pitch_table, beat_table, dur_table,
           W_l, W_r, b):
    pitch = x[:, 2]
    dur = x[:, 3]
    code = pitch * 128 + dur
    src = edge_index[0]
    dst = edge_index[1]
    srcp = jnp.concatenate([src, jnp.zeros((EP - E,), jnp.int32)]).reshape(
        EP // 128, 128)
    dstp = jnp.concatenate([dst, jnp.full((EP - E,), N, jnp.int32)]).reshape(
        EP // 128, 128)

    counts = _sc_hist_call(srcp, dstp, code).reshape(NP, RW)

    p2 = _pad_ids(pitch)
    d2 = _pad_ids(dur)
    bi2 = _pad_ids(beat_info)
    pt72 = jnp.pad(pitch_table[:66], ((0, 6), (0, 0)))
    dt72 = jnp.pad(dur_table, ((0, 6), (0, 0)))
    bt72 = jnp.pad(beat_table, ((0, 6), (0, 0)))
    wcp, wcd, wp, wd, wb = _fold_call(pt72, dt72, bt72, W_l, W_r)

    return _main_call(counts, p2, d2, bi2, wcp, wcd, wp, wd, wb,
                      b.reshape(1, 128))


# speedup vs baseline: 17.7467x; 1.1207x over previous
import functools

import jax
import jax.numpy as jnp
from jax import lax
from jax.experimental import pallas as pl
from jax.experimental.pallas import tpu as pltpu
from jax.experimental.pallas import tpu_sc as plsc

N = 50000
E = 800000
RW = 72
ROWS = 25088
ALLOC = 25216
NP = 2 * ROWS
ER = E // 128
CH = 1024
NCH = 49
PER_SC = ROWS * RW
PER_TILE = PER_SC // 16
ZB = 2304
CR = CH // 128
BN = 512
GRID = NP // BN


def _sc_hist(ei_hbm, code_hbm, out_hbm,
             srcv, dstv, codes_v, addr_v, ones_v, hi_v, zbuf,
             code_sh, counts_sh, sem_g, sem_a):
    c = lax.axis_index("c")
    s = lax.axis_index("s")
    lo = c * ROWS

    for m in range(128 // 16):
        ones_v[pl.ds(m * 16, 16)] = jnp.ones((16,), jnp.int32)
        hi_v[pl.ds(m * 16, 16)] = jnp.full((16,), 65536, jnp.int32)

    def zfill(i, _):
        zbuf[pl.ds(i * 16, 16)] = jnp.zeros((16,), jnp.int32)
        return 0
    lax.fori_loop(0, ZB // 16, zfill, 0)

    zbase = s * PER_TILE
    def zcp(i, _):
        pltpu.sync_copy(zbuf, counts_sh.at[pl.ds(zbase + i * ZB, ZB)])
        return 0
    lax.fori_loop(0, PER_TILE // ZB, zcp, 0)

    @pl.when(s == 0)
    def _():
        pltpu.sync_copy(code_hbm, code_sh)
    plsc.subcore_barrier()

    def _drain_adds():
        for j in range(8):
            pltpu.make_async_copy(ones_v, counts_sh.at[addr_v.at[j]],
                                  sem_a).wait()
        for j in range(8, 16):
            pltpu.make_async_copy(hi_v, counts_sh.at[addr_v.at[j]],
                                  sem_a).wait()

    rstart = (s * ER) // 16
    rend = ((s + 1) * ER) // 16
    def chunk(g, _):
        rb = rstart + g * CR
        rb_c = jnp.minimum(rb, ER - CR)
        oc = rb_c * 128
        pltpu.sync_copy(ei_hbm.at[pl.ds(oc, CH)], srcv)
        pltpu.sync_copy(ei_hbm.at[pl.ds(E + oc, CH)], dstv)
        gcps = [pltpu.async_copy(code_sh.at[srcv.at[pl.ds(j * 128, 128)]],
                                 codes_v.at[pl.ds(j * 128, 128)], sem_g)
                for j in range(CR)]
        for cp in gcps:
            cp.wait()
        @pl.when(g > 0)
        def _():
            _drain_adds()
        for i in range(CH // 16):
            j, k = i // 8, (i % 8) * 16
            rg = rb_c + j
            pen = jnp.where((rg >= rb) & (rg < rend), 0, 10000000)
            cv = codes_v[pl.ds(j * 128 + k, 16)]
            dv = dstv[pl.ds(j * 128 + k, 16)]
            p = lax.shift_right_logical(cv, 7)
            dr = lax.bitwise_and(cv, 127)
            row = (dv - lo) + pen
            ok = (row >= 0) & (row < ROWS)
            rowt = jnp.where(ok, row, ROWS)
            base = rowt * RW
            addr_v[j, pl.ds(k, 16)] = base + p
            addr_v[j + 8, pl.ds(k, 16)] = base + dr
        for j in range(8):
            pltpu.async_copy(ones_v, counts_sh.at[addr_v.at[j]], sem_a,
                             add=True)
        for j in range(8, 16):
            pltpu.async_copy(hi_v, counts_sh.at[addr_v.at[j]], sem_a,
                             add=True)
        return 0
    lax.fori_loop(0, NCH, chunk, 0)
    _drain_adds()
    plsc.subcore_barrier()

    gout = c * PER_SC + s * PER_TILE
    pltpu.sync_copy(counts_sh.at[pl.ds(zbase, PER_TILE)],
                    out_hbm.at[pl.ds(gout, PER_TILE)])


_sc_hist_call = functools.partial(
    pl.kernel,
    mesh=plsc.VectorSubcoreMesh(core_axis_name="c", subcore_axis_name="s"),
    compiler_params=pltpu.CompilerParams(needs_layout_passes=False),
    out_type=jax.ShapeDtypeStruct((NP * RW,), jnp.int32),
    scratch_types=[
        pltpu.VMEM((CH,), jnp.int32),
        pltpu.VMEM((CH,), jnp.int32),
        pltpu.VMEM((CH,), jnp.int32),
        pltpu.VMEM((16, 128), jnp.int32),
        pltpu.VMEM((128,), jnp.int32),
        pltpu.VMEM((128,), jnp.int32),
        pltpu.VMEM((ZB,), jnp.int32),
        pltpu.VMEM_SHARED((N,), jnp.int32),
        pltpu.VMEM_SHARED((ALLOC * RW,), jnp.int32),
        pltpu.SemaphoreType.DMA,
        pltpu.SemaphoreType.DMA,
    ],
)(_sc_hist)


def _fold(pt_ref, dt_ref, wl_ref, wr_ref,
          wcp_ref, wcd_ref, wp_ref, wd_ref):
    pt = pt_ref[:]
    dt = dt_ref[:]
    wl = wl_ref[:]
    wr = wr_ref[:]
    wcp_ref[:] = jnp.dot(pt, wl[0:32, :], preferred_element_type=jnp.float32)
    wcd_ref[:] = jnp.dot(dt, wl[32:64, :], preferred_element_type=jnp.float32)
    wp_ref[:] = jnp.dot(pt, wr[0:32, :], preferred_element_type=jnp.float32)
    wd_ref[:] = jnp.dot(dt, wr[32:64, :], preferred_element_type=jnp.float32)


def _fold_call(pt72, dt72, W_l, W_r):
    w = jax.ShapeDtypeStruct((72, 128), jnp.float32)
    return pl.pallas_call(
        _fold, out_shape=[w, w, w, w],
    )(pt72, dt72, W_l, W_r)


def _main(cnt_ref, p_ref, d_ref, bi_ref, wcp_ref, wcd_ref, wp_ref, wd_ref,
          bt_ref, b_ref, o_ref):
    ci = cnt_ref[:]
    pc = lax.bitwise_and(ci, 65535).astype(jnp.float32)
    dc = lax.shift_right_logical(ci, 16).astype(jnp.float32)
    deg = jnp.sum(pc, axis=1, keepdims=True)
    inv = 1.0 / jnp.maximum(deg, 1.0)
    iot = lax.broadcasted_iota(jnp.int32, (BN, 72), 1)
    ohp = (p_ref[:] == iot).astype(jnp.float32)
    ohd = (d_ref[:] == iot).astype(jnp.float32)
    ohb = (bi_ref[:] == iot).astype(jnp.float32)
    left = (jnp.dot(pc * inv, wcp_ref[:], preferred_element_type=jnp.float32)
            + jnp.dot(dc * inv, wcd_ref[:], preferred_element_type=jnp.float32)
            + jnp.dot(ohp, wp_ref[:], preferred_element_type=jnp.float32)
            + jnp.dot(ohd, wd_ref[:], preferred_element_type=jnp.float32)
            + b_ref[:])
    ss = jnp.sum(left * left, axis=1, keepdims=True)
    left = left * jnp.minimum(lax.rsqrt(ss), 1e12)
    left = jnp.where(left > 0, left, 0.2 * left)
    o_ref[:, 0:128] = left
    o_ref[:, 128:160] = jnp.dot(ohb, bt_ref[:],
                                preferred_element_type=jnp.float32)


def _main_call(counts, p2, d2, b2, wcp, wcd, wp, wd, bt72, bias):
    full = lambda shp: pl.BlockSpec(shp, lambda i: (0, 0))
    return pl.pallas_call(
        _main,
        grid=(GRID,),
        in_specs=[
            pl.BlockSpec((BN, RW), lambda i: (i, 0)),
            pl.BlockSpec((BN, 1), lambda i: (i, 0)),
            pl.BlockSpec((BN, 1), lambda i: (i, 0)),
            pl.BlockSpec((BN, 1), lambda i: (i, 0)),
            full((72, 128)),
            full((72, 128)),
            full((72, 128)),
            full((72, 128)),
            full((72, 32)),
            full((1, 128)),
        ],
        out_specs=pl.BlockSpec((BN, 160), lambda i: (i, 0)),
        out_shape=jax.ShapeDtypeStruct((N, 160), jnp.float32),
    )(counts, p2, d2, b2, wcp, wcd, wp, wd, bt72, bias)


def _pad_ids(a):
    return jnp.pad(a, (0, NP - N)).reshape(NP, 1)


def kernel(x, beat_info, edge_index, pitch_table, beat_table, dur_table,
           W_l, W_r, b):
    pitch = x[:, 2]
    dur = x[:, 3]
    code = pitch * 128 + dur
    ei2 = edge_index.reshape(2 * E)

    counts = _sc_hist_call(ei2, code).reshape(NP, RW)

    p2 = _pad_ids(pitch)
    d2 = _pad_ids(dur)
    bi2 = _pad_ids(beat_info)
    pt72 = jnp.pad(pitch_table[:66], ((0, 6), (0, 0)))
    dt72 = jnp.pad(dur_table, ((0, 6), (0, 0)))
    bt72 = jnp.pad(beat_table, ((0, 6), (0, 0)))
    wcp, wcd, wp, wd = _fold_call(pt72, dt72, W_l, W_r)

    return _main_call(counts, p2, d2, bi2, wcp, wcd, wp, wd, bt72,
                      b.reshape(1, 128))
